# baseline (speedup 1.0000x reference)
import functools

import jax
import jax.numpy as jnp
from jax import lax
from jax.experimental import pallas as pl
from jax.experimental.pallas import tpu as pltpu
from jax.experimental.pallas import tpu_sc as plsc

BATCH = 4096
EMBED_DIM = 64
KG_LAMBDA = 1e-05

_NC, _NS = 2, 16
_NW = _NC * _NS
_BPW = BATCH // _NW


def _sc_gather(h2, r2, pos2, neg2, table2, rel2):
    row = jax.ShapeDtypeStruct((BATCH, 2 * EMBED_DIM), jnp.float32)

    @functools.partial(
        pl.kernel,
        mesh=plsc.VectorSubcoreMesh(core_axis_name="c", subcore_axis_name="s"),
        out_type=[row, row, row, row],
        scratch_types=[
            pltpu.VMEM((_BPW,), jnp.int32),
            pltpu.VMEM((_BPW, 2 * EMBED_DIM), jnp.float32),
            pltpu.SemaphoreType.DMA,
        ],
    )
    def k(h_hbm, r_hbm, pos_hbm, neg_hbm, tab_hbm, rel_hbm,
          out_h, out_r, out_pos, out_neg, idx_v, rows_v, sem):
        wid = lax.axis_index("s") * _NC + lax.axis_index("c")
        base = wid * _BPW
        for idx_hbm, tbl, out in ((h_hbm, tab_hbm, out_h),
                                  (r_hbm, rel_hbm, out_r),
                                  (pos_hbm, tab_hbm, out_pos),
                                  (neg_hbm, tab_hbm, out_neg)):
            pltpu.sync_copy(idx_hbm.at[pl.ds(base, _BPW)], idx_v)
            pltpu.async_copy(tbl.at[idx_v], rows_v, sem).wait()
            pltpu.sync_copy(rows_v, out.at[pl.ds(base, _BPW)])

    return k(h2, r2, pos2, neg2, table2, rel2)


def _pick(packed_ref, par_ref):
    lo = packed_ref[:, :EMBED_DIM]
    hi = packed_ref[:, EMBED_DIM:]
    return jnp.where(par_ref[...] > 0, hi, lo)


def _tc_body(h2_ref, r2_ref, pos2_ref, neg2_ref, hp_ref, rp_ref, pp_ref,
             np_ref, hw1_ref, hw2_ref, hb_ref, rw1_ref, rw2_ref, rb_ref,
             w_ref, out_ref):
    he = _pick(h2_ref, hp_ref)
    re = _pick(r2_ref, rp_ref)
    pos = _pick(pos2_ref, pp_ref)
    neg = _pick(neg2_ref, np_ref)
    a1 = jnp.sum(re * hw1_ref[...], axis=1, keepdims=True)
    a2 = jnp.sum(he * rw2_ref[...], axis=1, keepdims=True)
    b1 = jnp.sum(re * hw2_ref[...], axis=1, keepdims=True)
    b2 = jnp.sum(he * rw1_ref[...], axis=1, keepdims=True)
    cross_h = he * a1 + re * a2 + hb_ref[...]
    cross_r = he * b1 + re * b2 + rb_ref[...]
    w1 = w_ref[:EMBED_DIM, :]
    w2 = w_ref[EMBED_DIM:, :]
    pred = (jnp.dot(cross_h, w1, preferred_element_type=jnp.float32)
            + jnp.dot(cross_r, w2, preferred_element_type=jnp.float32))
    x = jnp.sum(pred * (pos - neg), axis=1, keepdims=True)
    nls = jnp.maximum(-x, 0.0) + jnp.log1p(jnp.exp(-jnp.abs(x)))
    kg_loss = jnp.sum(nls) / BATCH
    l2 = (jnp.sum(cross_h * cross_h) + jnp.sum(cross_r * cross_r)
          + jnp.sum(pos * pos) + jnp.sum(neg * neg)) / (2.0 * BATCH)
    out_ref[0, 0] = kg_loss + KG_LAMBDA * l2


def kernel(h, r, pos_t, neg_t, entity_user_embed, relation_embed,
           h_trans_w1, h_trans_w2, h_bias_b, r_trans_w1, r_trans_w2, r_bias_b,
           sem_trans_w):
    n_rows = entity_user_embed.shape[0]
    table2 = entity_user_embed.reshape(n_rows // 2, 2 * EMBED_DIM)
    rel2 = relation_embed.reshape(relation_embed.shape[0] // 2, 2 * EMBED_DIM)
    h = h.astype(jnp.int32)
    r = r.astype(jnp.int32)
    pos_t = pos_t.astype(jnp.int32)
    neg_t = neg_t.astype(jnp.int32)

    h2_e, r2_e, pos2_e, neg2_e = _sc_gather(
        lax.shift_right_logical(h, 1), lax.shift_right_logical(r, 1),
        lax.shift_right_logical(pos_t, 1), lax.shift_right_logical(neg_t, 1),
        table2, rel2)

    def par(ix):
        return (ix & 1).astype(jnp.float32).reshape(BATCH, 1)

    out = pl.pallas_call(
        _tc_body,
        out_shape=jax.ShapeDtypeStruct((1, 1), jnp.float32),
        in_specs=[pl.BlockSpec(memory_space=pltpu.VMEM)] * 15,
        out_specs=pl.BlockSpec(memory_space=pltpu.SMEM),
    )(h2_e, r2_e, pos2_e, neg2_e, par(h), par(r), par(pos_t), par(neg_t),
      h_trans_w1.reshape(1, EMBED_DIM), h_trans_w2.reshape(1, EMBED_DIM),
      h_bias_b.reshape(1, EMBED_DIM),
      r_trans_w1.reshape(1, EMBED_DIM), r_trans_w2.reshape(1, EMBED_DIM),
      r_bias_b.reshape(1, EMBED_DIM), sem_trans_w)
    return out[0, 0]


# speedup vs baseline: 1.5080x; 1.5080x over previous
import functools

import jax
import jax.numpy as jnp
from jax import lax
from jax.experimental import pallas as pl
from jax.experimental.pallas import tpu as pltpu
from jax.experimental.pallas import tpu_sc as plsc

BATCH = 4096
EMBED_DIM = 64
KG_LAMBDA = 1e-05

_NC, _NS = 2, 16
_NW = _NC * _NS
_BPW = BATCH // _NW
_SPLIT_A = 74880
_SPLIT_B = 149760
_TAIL = 240
_PACK_ROWS = _SPLIT_A + _TAIL // 2
_PACK_C = 4992
_PACK_STEPS = _SPLIT_A // _PACK_C


def _pack_body(tab_ref, tail_ref, out_ref, in_a, in_b, out_v,
               sem_a, sem_b, sem_o, sem_t):
    C = _PACK_C

    def start_in(g, slot):
        c0 = g * C
        pltpu.make_async_copy(tab_ref.at[:, pl.ds(c0, C)],
                              in_a.at[slot], sem_a.at[slot]).start()
        pltpu.make_async_copy(tab_ref.at[:, pl.ds(_SPLIT_A + c0, C)],
                              in_b.at[slot], sem_b.at[slot]).start()

    tail_copy = pltpu.make_async_copy(
        tail_ref, out_ref.at[pl.ds(_SPLIT_A, _TAIL // 2), :], sem_t)
    tail_copy.start()
    start_in(0, 0)
    for g in range(_PACK_STEPS):
        slot = g % 2
        if g + 1 < _PACK_STEPS:
            start_in(g + 1, 1 - slot)
        pltpu.make_async_copy(tab_ref.at[:, pl.ds(g * C, C)],
                              in_a.at[slot], sem_a.at[slot]).wait()
        pltpu.make_async_copy(tab_ref.at[:, pl.ds(_SPLIT_A + g * C, C)],
                              in_b.at[slot], sem_b.at[slot]).wait()
        if g >= 2:
            pltpu.make_async_copy(out_v.at[slot],
                                  out_ref.at[pl.ds((g - 2) * C, C), :],
                                  sem_o.at[slot]).wait()
        out_v[slot, :, :EMBED_DIM] = jnp.transpose(in_a[slot], (1, 0))
        out_v[slot, :, EMBED_DIM:] = jnp.transpose(in_b[slot], (1, 0))
        pltpu.make_async_copy(out_v.at[slot],
                              out_ref.at[pl.ds(g * C, C), :],
                              sem_o.at[slot]).start()
    for g in (_PACK_STEPS - 2, _PACK_STEPS - 1):
        slot = g % 2
        pltpu.make_async_copy(out_v.at[slot],
                              out_ref.at[pl.ds(g * C, C), :],
                              sem_o.at[slot]).wait()
    tail_copy.wait()


def _pack_table(table):
    tab_t = table.T
    tail2 = table[_SPLIT_B:].reshape(_TAIL // 2, 2 * EMBED_DIM)
    return pl.pallas_call(
        _pack_body,
        in_specs=[pl.BlockSpec(memory_space=pl.ANY)] * 2,
        out_specs=pl.BlockSpec(memory_space=pl.ANY),
        out_shape=jax.ShapeDtypeStruct((_PACK_ROWS, 2 * EMBED_DIM),
                                       jnp.float32),
        scratch_shapes=[
            pltpu.VMEM((2, EMBED_DIM, _PACK_C), jnp.float32),
            pltpu.VMEM((2, EMBED_DIM, _PACK_C), jnp.float32),
            pltpu.VMEM((2, _PACK_C, 2 * EMBED_DIM), jnp.float32),
            pltpu.SemaphoreType.DMA((2,)),
            pltpu.SemaphoreType.DMA((2,)),
            pltpu.SemaphoreType.DMA((2,)),
            pltpu.SemaphoreType.DMA,
        ],
    )(tab_t, tail2)


def _sc_gather(h2, r2, pos2, neg2, table2, rel2):
    row = jax.ShapeDtypeStruct((BATCH, 2 * EMBED_DIM), jnp.float32)

    @functools.partial(
        pl.kernel,
        mesh=plsc.VectorSubcoreMesh(core_axis_name="c", subcore_axis_name="s"),
        out_type=[row, row, row, row],
        scratch_types=[
            pltpu.VMEM((_BPW,), jnp.int32),
            pltpu.VMEM((_BPW, 2 * EMBED_DIM), jnp.float32),
            pltpu.SemaphoreType.DMA,
        ],
    )
    def k(h_hbm, r_hbm, pos_hbm, neg_hbm, tab_hbm, rel_hbm,
          out_h, out_r, out_pos, out_neg, idx_v, rows_v, sem):
        wid = lax.axis_index("s") * _NC + lax.axis_index("c")
        base = wid * _BPW
        for idx_hbm, tbl, out in ((h_hbm, tab_hbm, out_h),
                                  (r_hbm, rel_hbm, out_r),
                                  (pos_hbm, tab_hbm, out_pos),
                                  (neg_hbm, tab_hbm, out_neg)):
            pltpu.sync_copy(idx_hbm.at[pl.ds(base, _BPW)], idx_v)
            pltpu.async_copy(tbl.at[idx_v], rows_v, sem).wait()
            pltpu.sync_copy(rows_v, out.at[pl.ds(base, _BPW)])

    return k(h2, r2, pos2, neg2, table2, rel2)


def _pick(packed_ref, par_ref):
    lo = packed_ref[:, :EMBED_DIM]
    hi = packed_ref[:, EMBED_DIM:]
    return jnp.where(par_ref[...] > 0, hi, lo)


def _tc_body(h2_ref, r2_ref, pos2_ref, neg2_ref, hp_ref, rp_ref, pp_ref,
             np_ref, hw1_ref, hw2_ref, hb_ref, rw1_ref, rw2_ref, rb_ref,
             w_ref, out_ref):
    he = _pick(h2_ref, hp_ref)
    re = _pick(r2_ref, rp_ref)
    pos = _pick(pos2_ref, pp_ref)
    neg = _pick(neg2_ref, np_ref)
    a1 = jnp.sum(re * hw1_ref[...], axis=1, keepdims=True)
    a2 = jnp.sum(he * rw2_ref[...], axis=1, keepdims=True)
    b1 = jnp.sum(re * hw2_ref[...], axis=1, keepdims=True)
    b2 = jnp.sum(he * rw1_ref[...], axis=1, keepdims=True)
    cross_h = he * a1 + re * a2 + hb_ref[...]
    cross_r = he * b1 + re * b2 + rb_ref[...]
    w1 = w_ref[:EMBED_DIM, :]
    w2 = w_ref[EMBED_DIM:, :]
    pred = (jnp.dot(cross_h, w1, preferred_element_type=jnp.float32)
            + jnp.dot(cross_r, w2, preferred_element_type=jnp.float32))
    x = jnp.sum(pred * (pos - neg), axis=1, keepdims=True)
    nls = jnp.maximum(-x, 0.0) + jnp.log1p(jnp.exp(-jnp.abs(x)))
    kg_loss = jnp.sum(nls) / BATCH
    l2 = (jnp.sum(cross_h * cross_h) + jnp.sum(cross_r * cross_r)
          + jnp.sum(pos * pos) + jnp.sum(neg * neg)) / (2.0 * BATCH)
    out_ref[0, 0] = kg_loss + KG_LAMBDA * l2


def kernel(h, r, pos_t, neg_t, entity_user_embed, relation_embed,
           h_trans_w1, h_trans_w2, h_bias_b, r_trans_w1, r_trans_w2, r_bias_b,
           sem_trans_w):
    table2 = _pack_table(entity_user_embed)
    rel2 = relation_embed.reshape(relation_embed.shape[0] // 2, 2 * EMBED_DIM)
    h = h.astype(jnp.int32)
    r = r.astype(jnp.int32)
    pos_t = pos_t.astype(jnp.int32)
    neg_t = neg_t.astype(jnp.int32)

    def erow(ix):
        tail_row = _SPLIT_A + lax.shift_right_logical(ix - _SPLIT_B, 1)
        return jnp.where(ix < _SPLIT_A, ix,
                         jnp.where(ix < _SPLIT_B, ix - _SPLIT_A, tail_row))

    def eflag(ix):
        return jnp.where(ix < _SPLIT_A, 0,
                         jnp.where(ix < _SPLIT_B, 1, ix & 1))

    h2_e, r2_e, pos2_e, neg2_e = _sc_gather(
        erow(h), lax.shift_right_logical(r, 1),
        erow(pos_t), erow(neg_t), table2, rel2)

    def par(flag):
        return flag.astype(jnp.float32).reshape(BATCH, 1)

    out = pl.pallas_call(
        _tc_body,
        out_shape=jax.ShapeDtypeStruct((1, 1), jnp.float32),
        in_specs=[pl.BlockSpec(memory_space=pltpu.VMEM)] * 15,
        out_specs=pl.BlockSpec(memory_space=pltpu.SMEM),
    )(h2_e, r2_e, pos2_e, neg2_e,
      par(eflag(h)), par(r & 1),
      par(eflag(pos_t)), par(eflag(neg_t)),
      h_trans_w1.reshape(1, EMBED_DIM), h_trans_w2.reshape(1, EMBED_DIM),
      h_bias_b.reshape(1, EMBED_DIM),
      r_trans_w1.reshape(1, EMBED_DIM), r_trans_w2.reshape(1, EMBED_DIM),
      r_bias_b.reshape(1, EMBED_DIM), sem_trans_w)
    return out[0, 0]


# speedup vs baseline: 1.5146x; 1.0044x over previous
import functools

import jax
import jax.numpy as jnp
from jax import lax
from jax.experimental import pallas as pl
from jax.experimental.pallas import tpu as pltpu
from jax.experimental.pallas import tpu_sc as plsc

BATCH = 4096
EMBED_DIM = 64
KG_LAMBDA = 1e-05

_NC, _NS = 2, 16
_NW = _NC * _NS
_BPW = BATCH // _NW
_SPLIT_A = 74880
_SPLIT_B = 149760
_TAIL = 240
_PACK_ROWS = _SPLIT_A + _TAIL // 2
_PACK_C = 8320
_PACK_STEPS = _SPLIT_A // _PACK_C


def _pack_body(tab_ref, tail_ref, out_ref, in_a, in_b, out_v,
               sem_a, sem_b, sem_o, sem_t):
    C = _PACK_C

    def start_in(g, slot):
        c0 = g * C
        pltpu.make_async_copy(tab_ref.at[:, pl.ds(c0, C)],
                              in_a.at[slot], sem_a.at[slot]).start()
        pltpu.make_async_copy(tab_ref.at[:, pl.ds(_SPLIT_A + c0, C)],
                              in_b.at[slot], sem_b.at[slot]).start()

    tail_copy = pltpu.make_async_copy(
        tail_ref, out_ref.at[pl.ds(_SPLIT_A, _TAIL // 2), :], sem_t)
    tail_copy.start()
    start_in(0, 0)
    for g in range(_PACK_STEPS):
        slot = g % 2
        if g + 1 < _PACK_STEPS:
            start_in(g + 1, 1 - slot)
        pltpu.make_async_copy(tab_ref.at[:, pl.ds(g * C, C)],
                              in_a.at[slot], sem_a.at[slot]).wait()
        pltpu.make_async_copy(tab_ref.at[:, pl.ds(_SPLIT_A + g * C, C)],
                              in_b.at[slot], sem_b.at[slot]).wait()
        if g >= 2:
            pltpu.make_async_copy(out_v.at[slot],
                                  out_ref.at[pl.ds((g - 2) * C, C), :],
                                  sem_o.at[slot]).wait()
        out_v[slot, :, :EMBED_DIM] = jnp.transpose(in_a[slot], (1, 0))
        out_v[slot, :, EMBED_DIM:] = jnp.transpose(in_b[slot], (1, 0))
        pltpu.make_async_copy(out_v.at[slot],
                              out_ref.at[pl.ds(g * C, C), :],
                              sem_o.at[slot]).start()
    for g in (_PACK_STEPS - 2, _PACK_STEPS - 1):
        slot = g % 2
        pltpu.make_async_copy(out_v.at[slot],
                              out_ref.at[pl.ds(g * C, C), :],
                              sem_o.at[slot]).wait()
    tail_copy.wait()


def _pack_table(table):
    tab_t = table.T
    tail2 = table[_SPLIT_B:].reshape(_TAIL // 2, 2 * EMBED_DIM)
    return pl.pallas_call(
        _pack_body,
        in_specs=[pl.BlockSpec(memory_space=pl.ANY)] * 2,
        out_specs=pl.BlockSpec(memory_space=pl.ANY),
        out_shape=jax.ShapeDtypeStruct((_PACK_ROWS, 2 * EMBED_DIM),
                                       jnp.float32),
        scratch_shapes=[
            pltpu.VMEM((2, EMBED_DIM, _PACK_C), jnp.float32),
            pltpu.VMEM((2, EMBED_DIM, _PACK_C), jnp.float32),
            pltpu.VMEM((2, _PACK_C, 2 * EMBED_DIM), jnp.float32),
            pltpu.SemaphoreType.DMA((2,)),
            pltpu.SemaphoreType.DMA((2,)),
            pltpu.SemaphoreType.DMA((2,)),
            pltpu.SemaphoreType.DMA,
        ],
    )(tab_t, tail2)


def _sc_gather(h2, r2, pos2, neg2, table2, rel2):
    row = jax.ShapeDtypeStruct((BATCH, 2 * EMBED_DIM), jnp.float32)

    @functools.partial(
        pl.kernel,
        mesh=plsc.VectorSubcoreMesh(core_axis_name="c", subcore_axis_name="s"),
        out_type=[row, row, row, row],
        scratch_types=[
            pltpu.VMEM((4, _BPW), jnp.int32),
            pltpu.VMEM((4, _BPW, 2 * EMBED_DIM), jnp.float32),
            pltpu.SemaphoreType.DMA((4,)),
            pltpu.SemaphoreType.DMA((4,)),
            pltpu.SemaphoreType.DMA((4,)),
        ],
    )
    def k(h_hbm, r_hbm, pos_hbm, neg_hbm, tab_hbm, rel_hbm,
          out_h, out_r, out_pos, out_neg, idx_v, rows_v, sem_i, sem_g, sem_o):
        wid = lax.axis_index("s") * _NC + lax.axis_index("c")
        base = wid * _BPW
        streams = ((h_hbm, tab_hbm, out_h), (r_hbm, rel_hbm, out_r),
                   (pos_hbm, tab_hbm, out_pos), (neg_hbm, tab_hbm, out_neg))
        idx_cp = [pltpu.make_async_copy(s[0].at[pl.ds(base, _BPW)],
                                        idx_v.at[i], sem_i.at[i])
                  for i, s in enumerate(streams)]
        for c in idx_cp:
            c.start()
        gathers = []
        for i, (_, tbl, _o) in enumerate(streams):
            idx_cp[i].wait()
            gathers.append(pltpu.async_copy(tbl.at[idx_v.at[i]],
                                            rows_v.at[i], sem_g.at[i]))
        outs = []
        for i, (_, _t, out) in enumerate(streams):
            gathers[i].wait()
            outs.append(pltpu.make_async_copy(
                rows_v.at[i], out.at[pl.ds(base, _BPW)], sem_o.at[i]))
            outs[-1].start()
        for c in outs:
            c.wait()

    return k(h2, r2, pos2, neg2, table2, rel2)


_TB = 512


def _pick(packed_ref, flag):
    lo = packed_ref[:, :EMBED_DIM]
    hi = packed_ref[:, EMBED_DIM:]
    return jnp.where(flag > 0, hi, lo)


def _tc_body(h2_ref, r2_ref, pos2_ref, neg2_ref, fl_ref, hw1_ref, hw2_ref,
             hb_ref, rw1_ref, rw2_ref, rb_ref, w_ref, out_ref):
    he = _pick(h2_ref, fl_ref[:, 0:1])
    re = _pick(r2_ref, fl_ref[:, 1:2])
    pos = _pick(pos2_ref, fl_ref[:, 2:3])
    neg = _pick(neg2_ref, fl_ref[:, 3:4])
    a1 = jnp.sum(re * hw1_ref[...], axis=1, keepdims=True)
    a2 = jnp.sum(he * rw2_ref[...], axis=1, keepdims=True)
    b1 = jnp.sum(re * hw2_ref[...], axis=1, keepdims=True)
    b2 = jnp.sum(he * rw1_ref[...], axis=1, keepdims=True)
    cross_h = he * a1 + re * a2 + hb_ref[...]
    cross_r = he * b1 + re * b2 + rb_ref[...]
    w1 = w_ref[:EMBED_DIM, :]
    w2 = w_ref[EMBED_DIM:, :]
    pred = (jnp.dot(cross_h, w1, preferred_element_type=jnp.float32)
            + jnp.dot(cross_r, w2, preferred_element_type=jnp.float32))
    x = jnp.sum(pred * (pos - neg), axis=1, keepdims=True)
    nls = jnp.maximum(-x, 0.0) + jnp.log1p(jnp.exp(-jnp.abs(x)))
    l2 = (jnp.sum(cross_h * cross_h) + jnp.sum(cross_r * cross_r)
          + jnp.sum(pos * pos) + jnp.sum(neg * neg))
    part = jnp.sum(nls) / BATCH + l2 * (KG_LAMBDA / (2.0 * BATCH))

    @pl.when(pl.program_id(0) == 0)
    def _():
        out_ref[0, 0] = 0.0

    out_ref[0, 0] += part


def kernel(h, r, pos_t, neg_t, entity_user_embed, relation_embed,
           h_trans_w1, h_trans_w2, h_bias_b, r_trans_w1, r_trans_w2, r_bias_b,
           sem_trans_w):
    table2 = _pack_table(entity_user_embed)
    rel2 = relation_embed.reshape(relation_embed.shape[0] // 2, 2 * EMBED_DIM)
    h = h.astype(jnp.int32)
    r = r.astype(jnp.int32)
    pos_t = pos_t.astype(jnp.int32)
    neg_t = neg_t.astype(jnp.int32)

    def erow(ix):
        tail_row = _SPLIT_A + lax.shift_right_logical(ix - _SPLIT_B, 1)
        return jnp.where(ix < _SPLIT_A, ix,
                         jnp.where(ix < _SPLIT_B, ix - _SPLIT_A, tail_row))

    def eflag(ix):
        return jnp.where(ix < _SPLIT_A, 0,
                         jnp.where(ix < _SPLIT_B, 1, ix & 1))

    h2_e, r2_e, pos2_e, neg2_e = _sc_gather(
        erow(h), lax.shift_right_logical(r, 1),
        erow(pos_t), erow(neg_t), table2, rel2)

    flags = jnp.stack([eflag(h), r & 1, eflag(pos_t), eflag(neg_t)],
                      axis=1).astype(jnp.float32)

    row_spec = pl.BlockSpec((_TB, 2 * EMBED_DIM), lambda i: (i, 0))
    vec_spec = pl.BlockSpec((1, EMBED_DIM), lambda i: (0, 0))
    out = pl.pallas_call(
        _tc_body,
        grid=(BATCH // _TB,),
        out_shape=jax.ShapeDtypeStruct((1, 1), jnp.float32),
        in_specs=[row_spec, row_spec, row_spec, row_spec,
                  pl.BlockSpec((_TB, 4), lambda i: (i, 0)),
                  vec_spec, vec_spec, vec_spec, vec_spec, vec_spec, vec_spec,
                  pl.BlockSpec((2 * EMBED_DIM, EMBED_DIM), lambda i: (0, 0))],
        out_specs=pl.BlockSpec((1, 1), lambda i: (0, 0),
                               memory_space=pltpu.SMEM),
    )(h2_e, r2_e, pos2_e, neg2_e, flags,
      h_trans_w1.reshape(1, EMBED_DIM), h_trans_w2.reshape(1, EMBED_DIM),
      h_bias_b.reshape(1, EMBED_DIM),
      r_trans_w1.reshape(1, EMBED_DIM), r_trans_w2.reshape(1, EMBED_DIM),
      r_bias_b.reshape(1, EMBED_DIM), sem_trans_w)
    return out[0, 0]


# speedup vs baseline: 1.9808x; 1.3078x over previous
import functools

import jax
import jax.numpy as jnp
from jax import lax
from jax.experimental import pallas as pl
from jax.experimental.pallas import tpu as pltpu
from jax.experimental.pallas import tpu_sc as plsc

BATCH = 4096
EMBED_DIM = 64
N_REL = 32
KG_LAMBDA = 1e-05

_NC, _NS = 2, 16
_NW = _NC * _NS
_BPW = BATCH // _NW

_N_ROWS = 150000
_MAIN = 149760
_TAIL = _N_ROWS - _MAIN
_PACK_C = 8320
_PACK_STEPS = (_MAIN // 2) // _PACK_C
_TB = 512


def _pack_body(tab_ref, tail_ref, out_ref, in_a, in_b, out_v, tail_v,
               sem_a, sem_b, sem_o, sem_t):
    C = _PACK_C
    half = _MAIN // 2

    tail_in = pltpu.make_async_copy(tail_ref, tail_v, sem_t)
    tail_in.start()

    def start_in(g, slot):
        pltpu.make_async_copy(tab_ref.at[:, pl.ds(g * C, C)],
                              in_a.at[slot], sem_a.at[slot]).start()
        pltpu.make_async_copy(tab_ref.at[:, pl.ds(half + g * C, C)],
                              in_b.at[slot], sem_b.at[slot]).start()

    start_in(0, 0)
    for g in range(_PACK_STEPS):
        slot = g % 2
        if g + 1 < _PACK_STEPS:
            start_in(g + 1, 1 - slot)
        pltpu.make_async_copy(tab_ref.at[:, pl.ds(g * C, C)],
                              in_a.at[slot], sem_a.at[slot]).wait()
        pltpu.make_async_copy(tab_ref.at[:, pl.ds(half + g * C, C)],
                              in_b.at[slot], sem_b.at[slot]).wait()
        if g >= 2:
            pltpu.make_async_copy(out_v.at[slot],
                                  out_ref.at[pl.ds((g - 2) * C, C), :],
                                  sem_o.at[slot]).wait()
        out_v[slot, :, :EMBED_DIM] = jnp.transpose(in_a[slot], (1, 0))
        out_v[slot, :, EMBED_DIM:] = jnp.transpose(in_b[slot], (1, 0))
        pltpu.make_async_copy(out_v.at[slot],
                              out_ref.at[pl.ds(g * C, C), :],
                              sem_o.at[slot]).start()
    tail_in.wait()
    tail_out = pltpu.make_async_copy(
        tail_v, out_ref.at[pl.ds(half, _TAIL // 2), :], sem_t)
    tail_out.start()
    for g in (_PACK_STEPS - 2, _PACK_STEPS - 1):
        slot = g % 2
        pltpu.make_async_copy(out_v.at[slot],
                              out_ref.at[pl.ds(g * C, C), :],
                              sem_o.at[slot]).wait()
    tail_out.wait()


def _pack_table(table):
    tab_t = table.T
    tail2 = table[_MAIN:].reshape(_TAIL // 2, 2 * EMBED_DIM)
    packed = pl.pallas_call(
        _pack_body,
        in_specs=[pl.BlockSpec(memory_space=pl.ANY)] * 2,
        out_specs=pl.BlockSpec(memory_space=pl.ANY),
        out_shape=jax.ShapeDtypeStruct((_N_ROWS // 2, 2 * EMBED_DIM),
                                       jnp.float32),
        scratch_shapes=[
            pltpu.VMEM((2, EMBED_DIM, _PACK_C), jnp.float32),
            pltpu.VMEM((2, EMBED_DIM, _PACK_C), jnp.float32),
            pltpu.VMEM((2, _PACK_C, 2 * EMBED_DIM), jnp.float32),
            pltpu.VMEM((_TAIL // 2, 2 * EMBED_DIM), jnp.float32),
            pltpu.SemaphoreType.DMA((2,)),
            pltpu.SemaphoreType.DMA((2,)),
            pltpu.SemaphoreType.DMA((2,)),
            pltpu.SemaphoreType.DMA,
        ],
    )(tab_t, tail2)
    return packed.reshape(_N_ROWS, EMBED_DIM)


def _sc_gather(h, pos_t, neg_t, table):
    row = jax.ShapeDtypeStruct((BATCH, EMBED_DIM), jnp.float32)

    @functools.partial(
        pl.kernel,
        mesh=plsc.VectorSubcoreMesh(core_axis_name="c", subcore_axis_name="s"),
        out_type=[row, row, row],
        scratch_types=[
            pltpu.VMEM((3, _BPW), jnp.int32),
            pltpu.VMEM((3, _BPW, EMBED_DIM), jnp.float32),
            pltpu.SemaphoreType.DMA((3,)),
            pltpu.SemaphoreType.DMA((3,)),
            pltpu.SemaphoreType.DMA((3,)),
        ],
        compiler_params=pltpu.CompilerParams(use_tc_tiling_on_sc=False),
    )
    def k(h_hbm, pos_hbm, neg_hbm, tab_hbm, out_h, out_pos, out_neg,
          idx_v, rows_v, sem_i, sem_g, sem_o):
        wid = lax.axis_index("s") * _NC + lax.axis_index("c")
        base = wid * _BPW
        streams = ((h_hbm, out_h), (pos_hbm, out_pos), (neg_hbm, out_neg))
        idx_cp = [pltpu.make_async_copy(s[0].at[pl.ds(base, _BPW)],
                                        idx_v.at[i], sem_i.at[i])
                  for i, s in enumerate(streams)]
        for c in idx_cp:
            c.start()
        gathers = []
        for i, _ in enumerate(streams):
            idx_cp[i].wait()
            gathers.append(pltpu.async_copy(tab_hbm.at[idx_v.at[i]],
                                            rows_v.at[i], sem_g.at[i]))
        outs = []
        for i, (_, out) in enumerate(streams):
            gathers[i].wait()
            outs.append(pltpu.make_async_copy(
                rows_v.at[i], out.at[pl.ds(base, _BPW)], sem_o.at[i]))
            outs[-1].start()
        for c in outs:
            c.wait()

    return k(h, pos_t, neg_t, table)


def _tc_body(h_ref, pos_ref, neg_ref, rf_ref, rel_ref, hw1_ref, hw2_ref,
             hb_ref, rw1_ref, rw2_ref, rb_ref, w_ref, out_ref):
    he = h_ref[...]
    pos = pos_ref[...]
    neg = neg_ref[...]
    lanes = lax.broadcasted_iota(jnp.int32, (_TB, N_REL), 1)
    onehot = (lanes.astype(jnp.float32) == rf_ref[...]).astype(jnp.float32)
    re = jnp.dot(onehot, rel_ref[...], preferred_element_type=jnp.float32)
    a1 = jnp.sum(re * hw1_ref[...], axis=1, keepdims=True)
    a2 = jnp.sum(he * rw2_ref[...], axis=1, keepdims=True)
    b1 = jnp.sum(re * hw2_ref[...], axis=1, keepdims=True)
    b2 = jnp.sum(he * rw1_ref[...], axis=1, keepdims=True)
    cross_h = he * a1 + re * a2 + hb_ref[...]
    cross_r = he * b1 + re * b2 + rb_ref[...]
    w1 = w_ref[:EMBED_DIM, :]
    w2 = w_ref[EMBED_DIM:, :]
    pred = (jnp.dot(cross_h, w1, preferred_element_type=jnp.float32)
            + jnp.dot(cross_r, w2, preferred_element_type=jnp.float32))
    x = jnp.sum(pred * (pos - neg), axis=1, keepdims=True)
    nls = jnp.maximum(-x, 0.0) + jnp.log1p(jnp.exp(-jnp.abs(x)))
    l2 = (jnp.sum(cross_h * cross_h) + jnp.sum(cross_r * cross_r)
          + jnp.sum(pos * pos) + jnp.sum(neg * neg))
    part = jnp.sum(nls) / BATCH + l2 * (KG_LAMBDA / (2.0 * BATCH))

    @pl.when(pl.program_id(0) == 0)
    def _():
        out_ref[0, 0] = 0.0

    out_ref[0, 0] += part


def kernel(h, r, pos_t, neg_t, entity_user_embed, relation_embed,
           h_trans_w1, h_trans_w2, h_bias_b, r_trans_w1, r_trans_w2, r_bias_b,
           sem_trans_w):
    table = _pack_table(entity_user_embed)

    def erow(ix):
        half = _MAIN // 2
        return jnp.where(ix < half, 2 * ix,
                         jnp.where(ix < _MAIN, 2 * (ix - half) + 1, ix))

    h_e, pos_e, neg_e = _sc_gather(
        erow(h.astype(jnp.int32)), erow(pos_t.astype(jnp.int32)),
        erow(neg_t.astype(jnp.int32)), table)

    r_f = r.astype(jnp.float32).reshape(BATCH, 1)

    row_spec = pl.BlockSpec((_TB, EMBED_DIM), lambda i: (i, 0))
    vec_spec = pl.BlockSpec((1, EMBED_DIM), lambda i: (0, 0))
    out = pl.pallas_call(
        _tc_body,
        grid=(BATCH // _TB,),
        out_shape=jax.ShapeDtypeStruct((1, 1), jnp.float32),
        in_specs=[row_spec, row_spec, row_spec,
                  pl.BlockSpec((_TB, 1), lambda i: (i, 0)),
                  pl.BlockSpec((N_REL, EMBED_DIM), lambda i: (0, 0)),
                  vec_spec, vec_spec, vec_spec, vec_spec, vec_spec, vec_spec,
                  pl.BlockSpec((2 * EMBED_DIM, EMBED_DIM), lambda i: (0, 0))],
        out_specs=pl.BlockSpec((1, 1), lambda i: (0, 0),
                               memory_space=pltpu.SMEM),
    )(h_e, pos_e, neg_e, r_f, relation_embed,
      h_trans_w1.reshape(1, EMBED_DIM), h_trans_w2.reshape(1, EMBED_DIM),
      h_bias_b.reshape(1, EMBED_DIM),
      r_trans_w1.reshape(1, EMBED_DIM), r_trans_w2.reshape(1, EMBED_DIM),
      r_bias_b.reshape(1, EMBED_DIM), sem_trans_w)
    return out[0, 0]


# speedup vs baseline: 2.1305x; 1.0756x over previous
import functools

import jax
import jax.numpy as jnp
from jax import lax
from jax.experimental import pallas as pl
from jax.experimental.pallas import tpu as pltpu
from jax.experimental.pallas import tpu_sc as plsc

BATCH = 4096
EMBED_DIM = 64
N_REL = 32
KG_LAMBDA = 1e-05

_NC, _NS = 2, 16
_NW = _NC * _NS
_BPW = BATCH // _NW

_N_ROWS = 150000
_MAIN = 149760
_TAIL = _N_ROWS - _MAIN
_PACK_C = 5760
_PACK_STEPS = (_MAIN // 2) // _PACK_C
_NBUF = 3
_TB = 1024


def _pack_body(tab_ref, tail_ref, out_ref, in_a, in_b, out_v, tail_v,
               sem_a, sem_b, sem_o, sem_t):
    C = _PACK_C
    half = _MAIN // 2

    tail_in = pltpu.make_async_copy(tail_ref, tail_v, sem_t)
    tail_in.start()

    def start_in(g, slot):
        pltpu.make_async_copy(tab_ref.at[:, pl.ds(g * C, C)],
                              in_a.at[slot], sem_a.at[slot]).start()
        pltpu.make_async_copy(tab_ref.at[:, pl.ds(half + g * C, C)],
                              in_b.at[slot], sem_b.at[slot]).start()

    for g in range(min(_NBUF - 1, _PACK_STEPS)):
        start_in(g, g % _NBUF)
    for g in range(_PACK_STEPS):
        slot = g % _NBUF
        if g + _NBUF - 1 < _PACK_STEPS:
            start_in(g + _NBUF - 1, (g + _NBUF - 1) % _NBUF)
        pltpu.make_async_copy(tab_ref.at[:, pl.ds(g * C, C)],
                              in_a.at[slot], sem_a.at[slot]).wait()
        pltpu.make_async_copy(tab_ref.at[:, pl.ds(half + g * C, C)],
                              in_b.at[slot], sem_b.at[slot]).wait()
        if g >= _NBUF:
            pltpu.make_async_copy(out_v.at[slot],
                                  out_ref.at[pl.ds((g - _NBUF) * C, C), :],
                                  sem_o.at[slot]).wait()
        out_v[slot, :, :EMBED_DIM] = jnp.transpose(in_a[slot], (1, 0))
        out_v[slot, :, EMBED_DIM:] = jnp.transpose(in_b[slot], (1, 0))
        pltpu.make_async_copy(out_v.at[slot],
                              out_ref.at[pl.ds(g * C, C), :],
                              sem_o.at[slot]).start()
    tail_in.wait()
    tail_out = pltpu.make_async_copy(
        tail_v, out_ref.at[pl.ds(half, _TAIL // 2), :], sem_t)
    tail_out.start()
    for g in range(max(0, _PACK_STEPS - _NBUF), _PACK_STEPS):
        slot = g % _NBUF
        pltpu.make_async_copy(out_v.at[slot],
                              out_ref.at[pl.ds(g * C, C), :],
                              sem_o.at[slot]).wait()
    tail_out.wait()


def _pack_table(table):
    tab_t = table.T
    tail2 = table[_MAIN:].reshape(_TAIL // 2, 2 * EMBED_DIM)
    packed = pl.pallas_call(
        _pack_body,
        in_specs=[pl.BlockSpec(memory_space=pl.ANY)] * 2,
        out_specs=pl.BlockSpec(memory_space=pl.ANY),
        out_shape=jax.ShapeDtypeStruct((_N_ROWS // 2, 2 * EMBED_DIM),
                                       jnp.float32),
        scratch_shapes=[
            pltpu.VMEM((_NBUF, EMBED_DIM, _PACK_C), jnp.float32),
            pltpu.VMEM((_NBUF, EMBED_DIM, _PACK_C), jnp.float32),
            pltpu.VMEM((_NBUF, _PACK_C, 2 * EMBED_DIM), jnp.float32),
            pltpu.VMEM((_TAIL // 2, 2 * EMBED_DIM), jnp.float32),
            pltpu.SemaphoreType.DMA((_NBUF,)),
            pltpu.SemaphoreType.DMA((_NBUF,)),
            pltpu.SemaphoreType.DMA((_NBUF,)),
            pltpu.SemaphoreType.DMA,
        ],
    )(tab_t, tail2)
    return packed.reshape(_N_ROWS, EMBED_DIM)


def _sc_gather(idx3, table):
    row = jax.ShapeDtypeStruct((BATCH, EMBED_DIM), jnp.float32)

    @functools.partial(
        pl.kernel,
        mesh=plsc.VectorSubcoreMesh(core_axis_name="c", subcore_axis_name="s"),
        out_type=[row, row, row],
        scratch_types=[
            pltpu.VMEM((3, _BPW), jnp.int32),
            pltpu.VMEM((3, _BPW, EMBED_DIM), jnp.float32),
            pltpu.SemaphoreType.DMA((3,)),
            pltpu.SemaphoreType.DMA((3,)),
            pltpu.SemaphoreType.DMA((3,)),
        ],
        compiler_params=pltpu.CompilerParams(use_tc_tiling_on_sc=False),
    )
    def k(idx_hbm, tab_hbm, out_h, out_pos, out_neg,
          idx_v, rows_v, sem_i, sem_g, sem_o):
        wid = lax.axis_index("s") * _NC + lax.axis_index("c")
        base = wid * _BPW
        outs3 = (out_h, out_pos, out_neg)
        idx_cp = [pltpu.make_async_copy(idx_hbm.at[i, pl.ds(base, _BPW)],
                                        idx_v.at[i], sem_i.at[i])
                  for i in range(3)]
        for c in idx_cp:
            c.start()
        gathers = []
        for i in range(3):
            idx_cp[i].wait()
            gathers.append(pltpu.async_copy(tab_hbm.at[idx_v.at[i]],
                                            rows_v.at[i], sem_g.at[i]))
        outs = []
        for i, out in enumerate(outs3):
            gathers[i].wait()
            outs.append(pltpu.make_async_copy(
                rows_v.at[i], out.at[pl.ds(base, _BPW)], sem_o.at[i]))
            outs[-1].start()
        for c in outs:
            c.wait()

    return k(idx3, table)


def _tc_body(h_ref, pos_ref, neg_ref, rf_ref, rel_ref, hw1_ref, hw2_ref,
             hb_ref, rw1_ref, rw2_ref, rb_ref, w_ref, out_ref):
    he = h_ref[...]
    pos = pos_ref[...]
    neg = neg_ref[...]
    lanes = lax.broadcasted_iota(jnp.int32, (_TB, N_REL), 1)
    onehot = (lanes.astype(jnp.float32) == rf_ref[...]).astype(jnp.float32)
    re = jnp.dot(onehot, rel_ref[...], preferred_element_type=jnp.float32)
    a1 = jnp.sum(re * hw1_ref[...], axis=1, keepdims=True)
    a2 = jnp.sum(he * rw2_ref[...], axis=1, keepdims=True)
    b1 = jnp.sum(re * hw2_ref[...], axis=1, keepdims=True)
    b2 = jnp.sum(he * rw1_ref[...], axis=1, keepdims=True)
    cross_h = he * a1 + re * a2 + hb_ref[...]
    cross_r = he * b1 + re * b2 + rb_ref[...]
    w1 = w_ref[:EMBED_DIM, :]
    w2 = w_ref[EMBED_DIM:, :]
    pred = (jnp.dot(cross_h, w1, preferred_element_type=jnp.float32)
            + jnp.dot(cross_r, w2, preferred_element_type=jnp.float32))
    x = jnp.sum(pred * (pos - neg), axis=1, keepdims=True)
    nls = jnp.maximum(-x, 0.0) + jnp.log1p(jnp.exp(-jnp.abs(x)))
    l2 = (jnp.sum(cross_h * cross_h) + jnp.sum(cross_r * cross_r)
          + jnp.sum(pos * pos) + jnp.sum(neg * neg))
    part = jnp.sum(nls) / BATCH + l2 * (KG_LAMBDA / (2.0 * BATCH))

    @pl.when(pl.program_id(0) == 0)
    def _():
        out_ref[0, 0] = 0.0

    out_ref[0, 0] += part


def kernel(h, r, pos_t, neg_t, entity_user_embed, relation_embed,
           h_trans_w1, h_trans_w2, h_bias_b, r_trans_w1, r_trans_w2, r_bias_b,
           sem_trans_w):
    table = _pack_table(entity_user_embed)

    def erow(ix):
        half = _MAIN // 2
        return jnp.where(ix < half, 2 * ix,
                         jnp.where(ix < _MAIN, 2 * (ix - half) + 1, ix))

    idx3 = erow(jnp.stack([h, pos_t, neg_t]).astype(jnp.int32))
    h_e, pos_e, neg_e = _sc_gather(idx3, table)

    r_f = r.astype(jnp.float32).reshape(BATCH, 1)

    row_spec = pl.BlockSpec((_TB, EMBED_DIM), lambda i: (i, 0))
    vec_spec = pl.BlockSpec((1, EMBED_DIM), lambda i: (0, 0))
    out = pl.pallas_call(
        _tc_body,
        grid=(BATCH // _TB,),
        out_shape=jax.ShapeDtypeStruct((1, 1), jnp.float32),
        in_specs=[row_spec, row_spec, row_spec,
                  pl.BlockSpec((_TB, 1), lambda i: (i, 0)),
                  pl.BlockSpec((N_REL, EMBED_DIM), lambda i: (0, 0)),
                  vec_spec, vec_spec, vec_spec, vec_spec, vec_spec, vec_spec,
                  pl.BlockSpec((2 * EMBED_DIM, EMBED_DIM), lambda i: (0, 0))],
        out_specs=pl.BlockSpec((1, 1), lambda i: (0, 0),
                               memory_space=pltpu.SMEM),
    )(h_e, pos_e, neg_e, r_f, relation_embed,
      h_trans_w1.reshape(1, EMBED_DIM), h_trans_w2.reshape(1, EMBED_DIM),
      h_bias_b.reshape(1, EMBED_DIM),
      r_trans_w1.reshape(1, EMBED_DIM), r_trans_w2.reshape(1, EMBED_DIM),
      r_bias_b.reshape(1, EMBED_DIM), sem_trans_w)
    return out[0, 0]


# speedup vs baseline: 2.1555x; 1.0117x over previous
import functools

import jax
import jax.numpy as jnp
from jax import lax
from jax.experimental import pallas as pl
from jax.experimental.pallas import tpu as pltpu
from jax.experimental.pallas import tpu_sc as plsc

BATCH = 4096
EMBED_DIM = 64
N_REL = 32
KG_LAMBDA = 1e-05

_NC, _NS = 2, 16
_NW = _NC * _NS
_BPW = BATCH // _NW

_N_ROWS = 150000
_MAIN = 149760
_TAIL = _N_ROWS - _MAIN
_PACK_C = 14976
_PACK_STEPS = (_MAIN // 2) // _PACK_C
_NBUF = 3
_TB = 1024


def _pack_body(tab_ref, tail_ref, out_ref, in_a, in_b, out_v, tail_v,
               sem_a, sem_b, sem_o, sem_t):
    C = _PACK_C
    half = _MAIN // 2

    tail_in = pltpu.make_async_copy(tail_ref, tail_v, sem_t)
    tail_in.start()

    def start_in(g, slot):
        pltpu.make_async_copy(tab_ref.at[:, pl.ds(g * C, C)],
                              in_a.at[slot], sem_a.at[slot]).start()
        pltpu.make_async_copy(tab_ref.at[:, pl.ds(half + g * C, C)],
                              in_b.at[slot], sem_b.at[slot]).start()

    for g in range(min(_NBUF - 1, _PACK_STEPS)):
        start_in(g, g % _NBUF)
    for g in range(_PACK_STEPS):
        slot = g % _NBUF
        if g + _NBUF - 1 < _PACK_STEPS:
            start_in(g + _NBUF - 1, (g + _NBUF - 1) % _NBUF)
        pltpu.make_async_copy(tab_ref.at[:, pl.ds(g * C, C)],
                              in_a.at[slot], sem_a.at[slot]).wait()
        pltpu.make_async_copy(tab_ref.at[:, pl.ds(half + g * C, C)],
                              in_b.at[slot], sem_b.at[slot]).wait()
        if g >= _NBUF:
            pltpu.make_async_copy(out_v.at[slot],
                                  out_ref.at[pl.ds((g - _NBUF) * C, C), :],
                                  sem_o.at[slot]).wait()
        out_v[slot, :, :EMBED_DIM] = jnp.transpose(in_a[slot], (1, 0))
        out_v[slot, :, EMBED_DIM:] = jnp.transpose(in_b[slot], (1, 0))
        pltpu.make_async_copy(out_v.at[slot],
                              out_ref.at[pl.ds(g * C, C), :],
                              sem_o.at[slot]).start()
    tail_in.wait()
    tail_out = pltpu.make_async_copy(
        tail_v, out_ref.at[pl.ds(half, _TAIL // 2), :], sem_t)
    tail_out.start()
    for g in range(max(0, _PACK_STEPS - _NBUF), _PACK_STEPS):
        slot = g % _NBUF
        pltpu.make_async_copy(out_v.at[slot],
                              out_ref.at[pl.ds(g * C, C), :],
                              sem_o.at[slot]).wait()
    tail_out.wait()


def _pack_table(table):
    tab_t = table.T
    tail2 = table[_MAIN:].reshape(_TAIL // 2, 2 * EMBED_DIM)
    packed = pl.pallas_call(
        _pack_body,
        in_specs=[pl.BlockSpec(memory_space=pl.ANY)] * 2,
        out_specs=pl.BlockSpec(memory_space=pl.ANY),
        out_shape=jax.ShapeDtypeStruct((_N_ROWS // 2, 2 * EMBED_DIM),
                                       jnp.float32),
        scratch_shapes=[
            pltpu.VMEM((_NBUF, EMBED_DIM, _PACK_C), jnp.float32),
            pltpu.VMEM((_NBUF, EMBED_DIM, _PACK_C), jnp.float32),
            pltpu.VMEM((_NBUF, _PACK_C, 2 * EMBED_DIM), jnp.float32),
            pltpu.VMEM((_TAIL // 2, 2 * EMBED_DIM), jnp.float32),
            pltpu.SemaphoreType.DMA((_NBUF,)),
            pltpu.SemaphoreType.DMA((_NBUF,)),
            pltpu.SemaphoreType.DMA((_NBUF,)),
            pltpu.SemaphoreType.DMA,
        ],
    )(tab_t, tail2)
    return packed.reshape(_N_ROWS, EMBED_DIM)


def _sc_gather(idx3, table):
    row = jax.ShapeDtypeStruct((BATCH, EMBED_DIM), jnp.float32)

    @functools.partial(
        pl.kernel,
        mesh=plsc.VectorSubcoreMesh(core_axis_name="c", subcore_axis_name="s"),
        out_type=[row, row, row],
        scratch_types=[
            pltpu.VMEM((3, _BPW), jnp.int32),
            pltpu.VMEM((3, _BPW, EMBED_DIM), jnp.float32),
            pltpu.SemaphoreType.DMA((3,)),
            pltpu.SemaphoreType.DMA((3,)),
            pltpu.SemaphoreType.DMA((3,)),
        ],
        compiler_params=pltpu.CompilerParams(use_tc_tiling_on_sc=False),
    )
    def k(idx_hbm, tab_hbm, out_h, out_pos, out_neg,
          idx_v, rows_v, sem_i, sem_g, sem_o):
        wid = lax.axis_index("s") * _NC + lax.axis_index("c")
        base = wid * _BPW
        outs3 = (out_h, out_pos, out_neg)
        idx_cp = [pltpu.make_async_copy(idx_hbm.at[i, pl.ds(base, _BPW)],
                                        idx_v.at[i], sem_i.at[i])
                  for i in range(3)]
        for c in idx_cp:
            c.start()
        gathers = []
        for i in range(3):
            idx_cp[i].wait()
            gathers.append(pltpu.async_copy(tab_hbm.at[idx_v.at[i]],
                                            rows_v.at[i], sem_g.at[i]))
        outs = []
        for i, out in enumerate(outs3):
            gathers[i].wait()
            outs.append(pltpu.make_async_copy(
                rows_v.at[i], out.at[pl.ds(base, _BPW)], sem_o.at[i]))
            outs[-1].start()
        for c in outs:
            c.wait()

    return k(idx3, table)


def _tc_body(h_ref, pos_ref, neg_ref, rf_ref, rel_ref, hw1_ref, hw2_ref,
             hb_ref, rw1_ref, rw2_ref, rb_ref, w_ref, out_ref):
    D = EMBED_DIM
    he = h_ref[...]
    pos = pos_ref[...]
    neg = neg_ref[...]
    rel = rel_ref[...]
    lanes = lax.broadcasted_iota(jnp.int32, (_TB, N_REL), 1).astype(jnp.float32)

    def dot64(x, w_row):
        return jnp.sum(x * w_row, axis=1, keepdims=True)

    nls_sum = 0.0
    l2 = 0.0
    for k in (0, 1):
        sl = slice(k * D, (k + 1) * D)
        hek = he[:, sl]
        posk = pos[:, sl]
        negk = neg[:, sl]
        onehot = (lanes == rf_ref[:, k:k + 1]).astype(jnp.float32)
        rek = jnp.dot(onehot, rel, preferred_element_type=jnp.float32)
        a1 = dot64(rek, hw1_ref[...])
        a2 = dot64(hek, rw2_ref[...])
        b1 = dot64(rek, hw2_ref[...])
        b2 = dot64(hek, rw1_ref[...])
        cross_h = hek * a1 + rek * a2 + hb_ref[...]
        cross_r = hek * b1 + rek * b2 + rb_ref[...]
        pred = (jnp.dot(cross_h, w_ref[:D, :],
                        preferred_element_type=jnp.float32)
                + jnp.dot(cross_r, w_ref[D:, :],
                          preferred_element_type=jnp.float32))
        x = jnp.sum(pred * (posk - negk), axis=1, keepdims=True)
        nls_sum += jnp.sum(jnp.maximum(-x, 0.0)
                           + jnp.log1p(jnp.exp(-jnp.abs(x))))
        l2 += (jnp.sum(cross_h * cross_h) + jnp.sum(cross_r * cross_r)
               + jnp.sum(posk * posk) + jnp.sum(negk * negk))
    part = nls_sum / BATCH + l2 * (KG_LAMBDA / (2.0 * BATCH))

    @pl.when(pl.program_id(0) == 0)
    def _():
        out_ref[0, 0] = 0.0

    out_ref[0, 0] += part


def kernel(h, r, pos_t, neg_t, entity_user_embed, relation_embed,
           h_trans_w1, h_trans_w2, h_bias_b, r_trans_w1, r_trans_w2, r_bias_b,
           sem_trans_w):
    table = _pack_table(entity_user_embed)

    def erow(ix):
        half = _MAIN // 2
        return jnp.where(ix < half, 2 * ix,
                         jnp.where(ix < _MAIN, 2 * (ix - half) + 1, ix))

    idx3 = erow(jnp.stack([h, pos_t, neg_t]).astype(jnp.int32))
    h_e, pos_e, neg_e = _sc_gather(idx3, table)

    h_e2 = h_e.reshape(BATCH // 2, 2 * EMBED_DIM)
    pos_e2 = pos_e.reshape(BATCH // 2, 2 * EMBED_DIM)
    neg_e2 = neg_e.reshape(BATCH // 2, 2 * EMBED_DIM)
    r_f = r.astype(jnp.float32).reshape(BATCH // 2, 2)

    row_spec = pl.BlockSpec((_TB, 2 * EMBED_DIM), lambda i: (i, 0))
    vec_spec = pl.BlockSpec((1, EMBED_DIM), lambda i: (0, 0))
    out = pl.pallas_call(
        _tc_body,
        grid=(BATCH // 2 // _TB,),
        out_shape=jax.ShapeDtypeStruct((1, 1), jnp.float32),
        in_specs=[row_spec, row_spec, row_spec,
                  pl.BlockSpec((_TB, 2), lambda i: (i, 0)),
                  pl.BlockSpec((N_REL, EMBED_DIM), lambda i: (0, 0)),
                  vec_spec, vec_spec, vec_spec, vec_spec, vec_spec, vec_spec,
                  pl.BlockSpec((2 * EMBED_DIM, EMBED_DIM), lambda i: (0, 0))],
        out_specs=pl.BlockSpec((1, 1), lambda i: (0, 0),
                               memory_space=pltpu.SMEM),
    )(h_e2, pos_e2, neg_e2, r_f, relation_embed,
      h_trans_w1.reshape(1, EMBED_DIM), h_trans_w2.reshape(1, EMBED_DIM),
      h_bias_b.reshape(1, EMBED_DIM),
      r_trans_w1.reshape(1, EMBED_DIM), r_trans_w2.reshape(1, EMBED_DIM),
      r_bias_b.reshape(1, EMBED_DIM), sem_trans_w)
    return out[0, 0]
